# baseline (speedup 1.0000x reference)
import functools

import jax
import jax.numpy as jnp
from jax import lax
from jax.experimental import pallas as pl
from jax.experimental.pallas import tpu as pltpu
from jax.experimental.pallas import tpu_sc as plsc

B = 16384
F = 26
D = 32
LANES = 16
V = 1000000
SUPER = 128
NSUP = V * D // SUPER
NCOLS = 7808
COLS_PER_W = NCOLS // 32
TAILS = (V - NCOLS * SUPER) * D // SUPER

CHUNK = 16
NIDX = CHUNK * F
IDX_W = 104
NSTREAM = NIDX // IDX_W


def _transpose_kernel(num_workers):
    mesh = plsc.VectorSubcoreMesh(core_axis_name="c", subcore_axis_name="s")

    @functools.partial(
        pl.kernel,
        mesh=mesh,
        out_type=jax.ShapeDtypeStruct((NSUP, SUPER), jnp.float32),
        compiler_params=pltpu.CompilerParams(
            use_tc_tiling_on_sc=True, needs_layout_passes=False),
        scratch_types=[
            pltpu.VMEM((D, SUPER), jnp.float32),
            pltpu.VMEM((D, SUPER), jnp.float32),
            pltpu.VMEM((D, SUPER), jnp.float32),
            pltpu.VMEM((D, SUPER), jnp.float32),
            pltpu.VMEM((TAILS, SUPER), jnp.float32),
            pltpu.SemaphoreType.DMA,
            pltpu.SemaphoreType.DMA,
            pltpu.SemaphoreType.DMA,
            pltpu.SemaphoreType.DMA,
        ],
    )
    def tr_kernel(tab_t, tail_r, out_hbm, in0, in1, o0, o1, tl,
                  isem0, isem1, osem0, osem1):
        num_cores = lax.axis_size("c")
        wid = lax.axis_index("s") * num_cores + lax.axis_index("c")
        base = wid * COLS_PER_W

        ins = (in0, in1)
        outs = (o0, o1)
        isems = (isem0, isem1)
        osems = (osem0, osem1)
        iota = lax.iota(jnp.int32, LANES)
        base32 = iota * D

        @pl.when(wid == 0)
        def _():
            pltpu.sync_copy(tail_r, tl)
            pltpu.sync_copy(tl, out_hbm.at[pl.ds(NCOLS * D, TAILS)])

        def fire_in(ti, par):
            off = pl.multiple_of(ti * SUPER, SUPER)
            pltpu.async_copy(tab_t.at[:, pl.ds(off, SUPER)],
                             ins[par], isems[par])

        def wait_in(par):
            pltpu.make_async_copy(tab_t.at[:, pl.ds(0, SUPER)],
                                  ins[par], isems[par]).wait()

        def transpose(par):
            src = ins[par]
            dst = outs[par]
            for d in range(D):
                for c0 in range(0, SUPER, LANES):
                    v = src[d, pl.ds(c0, LANES)]
                    tmp = base32 + (c0 * D + d)
                    rows = lax.shift_right_logical(tmp, 7)
                    cols = tmp & (SUPER - 1)
                    plsc.store_scatter(dst, [rows, cols], v)

        def fire_out(ti, par):
            off = pl.multiple_of(ti * D, D)
            pltpu.async_copy(outs[par],
                             out_hbm.at[pl.ds(off, D)], osems[par])

        def wait_out(par):
            pltpu.make_async_copy(outs[par],
                                  out_hbm.at[pl.ds(0, D)], osems[par]).wait()

        def body(g, _):
            for p in range(2):
                i = g * 2 + p
                ti = base + i

                @pl.when(i + 1 < COLS_PER_W)
                def _():
                    fire_in(ti + 1, (p + 1) % 2)

                wait_in(p)
                @pl.when(g >= 1)
                def _():
                    wait_out(p)

                transpose(p)
                fire_out(ti, p)
            return _

        fire_in(base, 0)
        lax.fori_loop(0, COLS_PER_W // 2, body, None)
        wait_out(0)
        wait_out(1)

    return tr_kernel


def _gather_kernel(num_workers):
    rows_per_w = B // num_workers
    nchunks = rows_per_w // CHUNK

    mesh = plsc.VectorSubcoreMesh(core_axis_name="c", subcore_axis_name="s")

    @functools.partial(
        pl.kernel,
        mesh=mesh,
        out_type=jax.ShapeDtypeStruct((D, B), jnp.float32),
        compiler_params=pltpu.CompilerParams(
            use_tc_tiling_on_sc=True, needs_layout_passes=False),
        scratch_types=[
            pltpu.VMEM((F, rows_per_w), jnp.int32),
            pltpu.VMEM((NIDX,), jnp.int32),
            pltpu.VMEM((NIDX,), jnp.int32),
            pltpu.VMEM((NIDX,), jnp.int32),
            pltpu.VMEM((NIDX,), jnp.int32),
            pltpu.VMEM((NIDX, SUPER), jnp.float32),
            pltpu.VMEM((D, 8 * CHUNK), jnp.float32),
            pltpu.SemaphoreType.DMA,
        ],
    )
    def emb_kernel(feats_hbm, table_hbm, out_hbm, idx_v, sup0, sup1,
                   col0, col1, rows0, out_v, sem0):
        num_cores = lax.axis_size("c")
        wid = lax.axis_index("s") * num_cores + lax.axis_index("c")
        b0 = pl.multiple_of(wid * rows_per_w, SUPER)

        pltpu.sync_copy(feats_hbm.at[:, pl.ds(b0, rows_per_w)], idx_v)

        sups = (sup0, sup1)
        cols = (col0, col1)
        bufs = (rows0, rows0)
        sems = (sem0, sem0)

        def fire(k, par):
            sup = sups[par]
            col = cols[par]
            sem = sems[par]
            for f in range(F):
                raw = idx_v[f, pl.ds(k * CHUNK, CHUNK)]
                sup[pl.ds(f * CHUNK, CHUNK)] = lax.shift_right_logical(raw, 2)
                col[pl.ds(f * CHUNK, CHUNK)] = (raw & 3) * D
            for j in range(NSTREAM):
                pltpu.async_copy(
                    table_hbm.at[sup.at[pl.ds(j * IDX_W, IDX_W)]],
                    bufs[par].at[pl.ds(j * IDX_W, IDX_W)],
                    sem,
                )

        def drain(par):
            for j in range(NSTREAM):
                pltpu.make_async_copy(
                    table_hbm.at[pl.ds(0, IDX_W)],
                    bufs[par].at[pl.ds(j * IDX_W, IDX_W)],
                    sems[par],
                ).wait()

        def reduce(k, par):
            buf = bufs[par]
            col = cols[par]
            iota = lax.iota(jnp.int32, LANES)
            slot_off = (k % 8) * CHUNK
            dg = 8
            for d0 in range(0, D, dg):
                accs = [None] * dg
                for f in range(F):
                    rows = iota + (f * CHUNK)
                    cols0 = col[pl.ds(f * CHUNK, CHUNK)]
                    for i in range(dg):
                        v = plsc.load_gather(buf, [rows, cols0 + (d0 + i)])
                        accs[i] = v if f == 0 else accs[i] + v
                for i in range(dg):
                    out_v[d0 + i, pl.ds(slot_off, CHUNK)] = accs[i]

            @pl.when(k % 8 == 7)
            def _():
                boff = pl.multiple_of(b0 + (k - 7) * CHUNK, SUPER)
                pltpu.sync_copy(out_v, out_hbm.at[:, pl.ds(boff, 8 * CHUNK)])

        def body(k, _):
            fire(k, 0)
            drain(0)
            reduce(k, 0)
            return _

        lax.fori_loop(0, nchunks, body, None)

    return emb_kernel


def kernel(categorical_feats, table):
    info = plsc.get_sparse_core_info()
    num_workers = info.num_cores * info.num_subcores
    feats_t = categorical_feats.T.astype(jnp.int32)
    tab_t = table.T
    tail_r = table[NCOLS * SUPER:].reshape(TAILS, SUPER)
    table_r = _transpose_kernel(num_workers)(tab_t, tail_r)
    out_t = _gather_kernel(num_workers)(feats_t, table_r)
    return out_t.T


# speedup vs baseline: 1.2921x; 1.2921x over previous
import functools

import jax
import jax.numpy as jnp
from jax import lax
from jax.experimental import pallas as pl
from jax.experimental.pallas import tpu as pltpu
from jax.experimental.pallas import tpu_sc as plsc

B = 16384
F = 26
D = 32
LANES = 16
V = 1000000
SUPER = 128
NSUP = V * D // SUPER

BLOCK = 128
CHUNK = 16
NIDX = CHUNK * F
IDX_W = 104
NSTREAM = NIDX // IDX_W
CPB = BLOCK // CHUNK


def _make_kernel(num_workers):
    rows_per_w = B // num_workers
    nchunks = rows_per_w // CHUNK
    nblocks = rows_per_w // BLOCK

    mesh = plsc.VectorSubcoreMesh(core_axis_name="c", subcore_axis_name="s")

    @functools.partial(
        pl.kernel,
        mesh=mesh,
        out_type=jax.ShapeDtypeStruct((D, B), jnp.float32),
        compiler_params=pltpu.CompilerParams(
            use_tc_tiling_on_sc=True, needs_layout_passes=False),
        scratch_types=[
            pltpu.VMEM((F, BLOCK), jnp.int32),
            pltpu.VMEM((NIDX,), jnp.int32),
            pltpu.VMEM((NIDX,), jnp.int32),
            pltpu.VMEM((NIDX,), jnp.int32),
            pltpu.VMEM((NIDX,), jnp.int32),
            pltpu.VMEM((NIDX, SUPER), jnp.float32),
            pltpu.VMEM((NIDX, SUPER), jnp.float32),
            pltpu.VMEM((D, BLOCK), jnp.float32),
            pltpu.SemaphoreType.DMA,
            pltpu.SemaphoreType.DMA,
        ],
    )
    def emb_kernel(feats_hbm, table_hbm, out_hbm, idx_v, sup0, sup1,
                   col0, col1, rows0, rows1, out_v, sem0, sem1):
        num_cores = lax.axis_size("c")
        wid = lax.axis_index("s") * num_cores + lax.axis_index("c")
        b0 = pl.multiple_of(wid * rows_per_w, BLOCK)

        sups = (sup0, sup1)
        cols = (col0, col1)
        bufs = (rows0, rows1)
        sems = (sem0, sem1)

        def stage_idx(m):
            off = pl.multiple_of(b0 + m * BLOCK, BLOCK)
            pltpu.sync_copy(feats_hbm.at[:, pl.ds(off, BLOCK)], idx_v)

        def fire(k, par):
            idx = idx_v
            sup = sups[par]
            col = cols[par]
            sem = sems[par]
            c_in_b = k % CPB
            for f in range(F):
                raw = idx[f, pl.ds(c_in_b * CHUNK, CHUNK)]
                sup[pl.ds(f * CHUNK, CHUNK)] = lax.shift_right_logical(raw, 2)
                col[pl.ds(f * CHUNK, CHUNK)] = (raw & 3) * D
            for j in range(NSTREAM):
                pltpu.async_copy(
                    table_hbm.at[sup.at[pl.ds(j * IDX_W, IDX_W)]],
                    bufs[par].at[pl.ds(j * IDX_W, IDX_W)],
                    sem,
                )

        def drain(par):
            for j in range(NSTREAM):
                pltpu.make_async_copy(
                    table_hbm.at[pl.ds(0, IDX_W)],
                    bufs[par].at[pl.ds(j * IDX_W, IDX_W)],
                    sems[par],
                ).wait()

        def reduce(k, par):
            buf = bufs[par]
            col = cols[par]
            iota = lax.iota(jnp.int32, LANES)
            slot_off = (k % CPB) * CHUNK
            dg = 8
            zero = jnp.zeros((LANES,), jnp.float32)
            for d0 in range(0, D, dg):
                def fbody(f, accs):
                    rows = iota + f * CHUNK
                    cols0 = col[pl.ds(f * CHUNK, CHUNK)]
                    return tuple(
                        accs[i] + plsc.load_gather(buf, [rows, cols0 + (d0 + i)])
                        for i in range(dg))

                accs = lax.fori_loop(0, F, fbody, (zero,) * dg)
                for i in range(dg):
                    out_v[d0 + i, pl.ds(slot_off, CHUNK)] = accs[i]

            @pl.when(k % CPB == CPB - 1)
            def _():
                boff = pl.multiple_of(b0 + (k + 1 - CPB) * CHUNK, BLOCK)
                pltpu.sync_copy(out_v, out_hbm.at[:, pl.ds(boff, BLOCK)])

        def body(g, _):
            for p in range(2):
                k = g * 2 + p

                @pl.when(k < nchunks - 1)
                def _():
                    knext = k + 1

                    @pl.when(knext % CPB == 0)
                    def _():
                        stage_idx(knext // CPB)

                    fire(knext, (p + 1) % 2)

                drain(p)
                reduce(k, p)
            return _

        stage_idx(0)
        fire(0, 0)
        lax.fori_loop(0, nchunks // 2, body, None)

    return emb_kernel


def kernel(categorical_feats, table):
    info = plsc.get_sparse_core_info()
    num_workers = info.num_cores * info.num_subcores
    feats_t = categorical_feats.T.astype(jnp.int32)
    table_r = table.reshape(NSUP, SUPER)
    out_t = _make_kernel(num_workers)(feats_t, table_r)
    return out_t.T


# speedup vs baseline: 1.6677x; 1.2907x over previous
import functools

import jax
import jax.numpy as jnp
from jax import lax
from jax.experimental import pallas as pl
from jax.experimental.pallas import tpu as pltpu
from jax.experimental.pallas import tpu_sc as plsc

B = 16384
F = 26
D = 32
LANES = 16

CHUNK = 64


def _make_kernel(num_workers):
    rows_per_w = B // num_workers
    nchunks = rows_per_w // CHUNK

    mesh = plsc.VectorSubcoreMesh(core_axis_name="c", subcore_axis_name="s")

    @functools.partial(
        pl.kernel,
        mesh=mesh,
        out_type=jax.ShapeDtypeStruct((B, D), jnp.float32),
        compiler_params=pltpu.CompilerParams(use_tc_tiling_on_sc=False),
        scratch_types=[
            pltpu.VMEM((F, rows_per_w), jnp.int32),
            pltpu.VMEM((F, CHUNK, D), jnp.float32),
            pltpu.VMEM((F, CHUNK, D), jnp.float32),
            pltpu.VMEM((CHUNK, D), jnp.float32),
            pltpu.SemaphoreType.DMA,
            pltpu.SemaphoreType.DMA,
        ],
    )
    def emb_kernel(feats_hbm, table_hbm, out_hbm, idx_v, rows0, rows1, out_v,
                   sem0, sem1):
        num_cores = lax.axis_size("c")
        wid = lax.axis_index("s") * num_cores + lax.axis_index("c")
        b0 = wid * rows_per_w

        pltpu.sync_copy(feats_hbm.at[:, pl.ds(b0, rows_per_w)], idx_v)

        bufs = (rows0, rows1)
        sems = (sem0, sem1)

        def fire(k):
            buf = bufs[k % 2]
            sem = sems[k % 2]
            cs = []
            for f in range(F):
                cs.append(
                    pltpu.async_copy(
                        table_hbm.at[idx_v.at[f, pl.ds(k * CHUNK, CHUNK)]],
                        buf.at[f],
                        sem,
                    )
                )
            return cs

        inflight = fire(0)
        for k in range(nchunks):
            buf = bufs[k % 2]
            nxt = fire(k + 1) if k + 1 < nchunks else []
            for c in inflight:
                c.wait()
            inflight = nxt

            def reduce_row(j, _):
                lo = buf[0, j, pl.ds(0, LANES)]
                hi = buf[0, j, pl.ds(LANES, LANES)]
                for f in range(1, F):
                    lo = lo + buf[f, j, pl.ds(0, LANES)]
                    hi = hi + buf[f, j, pl.ds(LANES, LANES)]
                out_v[j, pl.ds(0, LANES)] = lo
                out_v[j, pl.ds(LANES, LANES)] = hi
                return _

            lax.fori_loop(0, CHUNK, reduce_row, None)

            pltpu.sync_copy(out_v, out_hbm.at[pl.ds(b0 + k * CHUNK, CHUNK)])

    return emb_kernel


def kernel(categorical_feats, table):
    info = plsc.get_sparse_core_info()
    num_workers = info.num_cores * info.num_subcores
    feats_t = categorical_feats.T.astype(jnp.int32)
    return _make_kernel(num_workers)(feats_t, table)
